# Initial kernel scaffold; baseline (speedup 1.0000x reference)
import jax
import jax.numpy as jnp
from jax.experimental import pallas as pl


def kernel(pref, risk, intensity, confidence, user_ids, attr_ids, polarity):
    raise NotImplementedError("write your pallas kernel here")


# speedup vs baseline: 1.0910x; 1.0910x over previous
import jax
import jax.numpy as jnp
from jax import lax
from jax.experimental import pallas as pl
from jax.experimental.pallas import tpu as pltpu
from jax.experimental.pallas import tpu_sc as plsc

U = 100000
A = 128
B = 16384
HALF = 64
NS = 16
CHUNK = B // NS
ROWS = 128

ETA_POS = 0.1
ETA_NEG = 1.0
ETA_DECAY = 0.2
RHO_ABSA = 0.75


def _coef_body(inten_ref, conf_ref, pol_ref, addp_ref, lm_ref, ar_ref):
    inten = inten_ref[...]
    conf = conf_ref[...]
    pol = pol_ref[...]
    d = inten * conf
    valid = conf >= RHO_ABSA
    pos = valid & (pol == 1)
    neg = valid & (pol == -1)
    zero = jnp.zeros_like(d)
    addp_ref[...] = jnp.where(pos, ETA_POS * d, zero)
    lm_ref[...] = jnp.where(pos, jnp.log1p(-ETA_DECAY * d), zero)
    ar_ref[...] = jnp.where(neg, ETA_NEG * d, zero)


def _sc_body(pref_hbm, risk_hbm, uid_hbm, kid_hbm, cap_hbm, clm_hbm, car_hbm,
             out_hbm,
             uids, kids, slots, jvec, cbuf,
             uidsc, slotc, ibuf, vbuf,
             pidx, oidx,
             pbuf, obuf, accrows,
             rep_s, acc_s):
    c = lax.axis_index("c")
    s = lax.axis_index("s")
    base = s * CHUNK
    chalf = c * HALF
    i16 = lax.iota(jnp.int32, 16)

    pltpu.sync_copy(uid_hbm.at[pl.ds(base, CHUNK)], uids)
    pltpu.sync_copy(kid_hbm.at[pl.ds(base, CHUNK)], kids)

    def _mk_jvec(g, carry):
        jvec[pl.ds(g * 16, 16)] = base + g * 16 + i16
        return carry
    lax.fori_loop(0, CHUNK // 16, _mk_jvec, 0)

    for q in range(CHUNK // 128):
        for g in range(8):
            uidsc[pl.ds(g * 16, 16)] = uids[pl.ds(q * 128 + g * 16, 16)]
        pltpu.sync_copy(jvec.at[pl.ds(q * 128, 128)], rep_s.at[uidsc])
    plsc.subcore_barrier()

    for q in range(CHUNK // 128):
        for g in range(8):
            uidsc[pl.ds(g * 16, 16)] = uids[pl.ds(q * 128 + g * 16, 16)]
        pltpu.sync_copy(rep_s.at[uidsc], slotc)
        for g in range(8):
            slots[pl.ds(q * 128 + g * 16, 16)] = slotc[pl.ds(g * 16, 16)]

    def zero_stripe():
        def _zero(g, carry):
            accrows[pl.ds(g * 16, 16)] = jnp.zeros((16,), jnp.float32)
            return carry
        lax.fori_loop(0, (ROWS * HALF) // 16, _zero, 0)
        for t in range(CHUNK // ROWS):
            pltpu.sync_copy(
                accrows, acc_s.at[pl.ds((base + t * ROWS) * HALF,
                                        ROWS * HALF)])

    def accumulate(coef_hbm):
        pltpu.sync_copy(coef_hbm.at[pl.ds(base, CHUNK)], cbuf)

        def _acc(q, carry):
            for g in range(8):
                k = kids[pl.ds(q * 128 + g * 16, 16)]
                sl = slots[pl.ds(q * 128 + g * 16, 16)]
                cv = cbuf[pl.ds(q * 128 + g * 16, 16)]
                kp = k - chalf
                inh = (kp >= 0) & (kp < HALF)
                kpc = jnp.clip(kp, 0, HALF - 1)
                ibuf[pl.ds(g * 16, 16)] = sl * HALF + kpc
                vbuf[pl.ds(g * 16, 16)] = jnp.where(
                    inh, cv, jnp.zeros_like(cv))
            pltpu.sync_copy(vbuf, acc_s.at[ibuf], add=True)
            return carry
        lax.fori_loop(0, CHUNK // 128, _acc, 0)

    def load_acc(t):
        pltpu.sync_copy(
            acc_s.at[pl.ds((base + t * ROWS) * HALF, ROWS * HALF)], accrows)

    zero_stripe()
    plsc.subcore_barrier()
    accumulate(clm_hbm)
    plsc.subcore_barrier()
    for t in range(CHUNK // ROWS):
        for g in range(8):
            pidx[pl.ds(g * 16, 16)] = uids[pl.ds(t * 128 + g * 16, 16)]
            jv = jvec[pl.ds(t * 128 + g * 16, 16)]
            oidx[pl.ds(g * 16, 16)] = jv * 2 + c
        pltpu.sync_copy(risk_hbm.at[pidx], pbuf)
        load_acc(t)

        def _row1(r, carry):
            for g in range(4):
                acc = accrows[pl.ds(r * HALF + g * 16, 16)]
                rv = pbuf[r, pl.ds(chalf + g * 16, 16)]
                obuf[r, pl.ds(HALF + g * 16, 16)] = rv * jnp.exp(acc)
            return carry
        lax.fori_loop(0, ROWS, _row1, 0)
        pltpu.sync_copy(obuf, out_hbm.at[oidx])

    zero_stripe()
    plsc.subcore_barrier()
    accumulate(car_hbm)
    plsc.subcore_barrier()
    for t in range(CHUNK // ROWS):
        for g in range(8):
            jv = jvec[pl.ds(t * 128 + g * 16, 16)]
            oidx[pl.ds(g * 16, 16)] = jv * 2 + c
        pltpu.sync_copy(out_hbm.at[oidx], obuf)
        load_acc(t)

        def _row2(r, carry):
            for g in range(4):
                acc = accrows[pl.ds(r * HALF + g * 16, 16)]
                rv = obuf[r, pl.ds(HALF + g * 16, 16)]
                obuf[r, pl.ds(HALF + g * 16, 16)] = jnp.clip(
                    rv + acc, 0.0, 1.0)
            return carry
        lax.fori_loop(0, ROWS, _row2, 0)
        pltpu.sync_copy(obuf, out_hbm.at[oidx])

    zero_stripe()
    plsc.subcore_barrier()
    accumulate(cap_hbm)
    plsc.subcore_barrier()
    for t in range(CHUNK // ROWS):
        for g in range(8):
            pidx[pl.ds(g * 16, 16)] = uids[pl.ds(t * 128 + g * 16, 16)]
            jv = jvec[pl.ds(t * 128 + g * 16, 16)]
            oidx[pl.ds(g * 16, 16)] = jv * 2 + c
        pltpu.sync_copy(out_hbm.at[oidx], obuf)
        pltpu.sync_copy(pref_hbm.at[pidx], pbuf)
        load_acc(t)

        def _row3(r, carry):
            for g in range(4):
                acc = accrows[pl.ds(r * HALF + g * 16, 16)]
                pv = pbuf[r, pl.ds(chalf + g * 16, 16)]
                obuf[r, pl.ds(g * 16, 16)] = jnp.clip(pv + acc, 0.0, 1.0)
            return carry
        lax.fori_loop(0, ROWS, _row3, 0)
        pltpu.sync_copy(obuf, out_hbm.at[oidx])
    plsc.subcore_barrier()

    for t in range(CHUNK // ROWS):
        for g in range(8):
            sl = slots[pl.ds(t * 128 + g * 16, 16)]
            jv = jvec[pl.ds(t * 128 + g * 16, 16)]
            pidx[pl.ds(g * 16, 16)] = sl * 2 + c
            oidx[pl.ds(g * 16, 16)] = jv * 2 + c
        pltpu.sync_copy(out_hbm.at[pidx], obuf)
        pltpu.sync_copy(obuf, out_hbm.at[oidx])


def kernel(pref, risk, intensity, confidence, user_ids, attr_ids, polarity):
    addp, lm, ar = pl.pallas_call(
        _coef_body,
        out_shape=(jax.ShapeDtypeStruct((128, 128), jnp.float32),) * 3,
    )(intensity.reshape(128, 128), confidence.reshape(128, 128),
      polarity.reshape(128, 128))

    mesh = plsc.VectorSubcoreMesh(core_axis_name="c", subcore_axis_name="s")
    fn = pl.kernel(
        _sc_body,
        out_type=jax.ShapeDtypeStruct((B * 2, A), jnp.float32),
        mesh=mesh,
        compiler_params=pltpu.CompilerParams(needs_layout_passes=False),
        scratch_types=[
            pltpu.VMEM((CHUNK,), jnp.int32),
            pltpu.VMEM((CHUNK,), jnp.int32),
            pltpu.VMEM((CHUNK,), jnp.int32),
            pltpu.VMEM((CHUNK,), jnp.int32),
            pltpu.VMEM((CHUNK,), jnp.float32),
            pltpu.VMEM((128,), jnp.int32),
            pltpu.VMEM((128,), jnp.int32),
            pltpu.VMEM((128,), jnp.int32),
            pltpu.VMEM((128,), jnp.float32),
            pltpu.VMEM((128,), jnp.int32),
            pltpu.VMEM((128,), jnp.int32),
            pltpu.VMEM((ROWS, A), jnp.float32),
            pltpu.VMEM((ROWS, A), jnp.float32),
            pltpu.VMEM((ROWS * HALF,), jnp.float32),
            pltpu.VMEM_SHARED((U,), jnp.int32),
            pltpu.VMEM_SHARED((B * HALF,), jnp.float32),
        ],
    )
    out2 = fn(pref, risk, user_ids, attr_ids,
              addp.reshape(B), lm.reshape(B), ar.reshape(B))
    return jnp.swapaxes(out2.reshape(B, 2, 2, HALF), 1, 2).reshape(B, 2 * A)


# speedup vs baseline: 1.4046x; 1.2874x over previous
import jax
import jax.numpy as jnp
from jax import lax
from jax.experimental import pallas as pl
from jax.experimental.pallas import tpu as pltpu
from jax.experimental.pallas import tpu_sc as plsc

U = 100000
A = 128
B = 16384
HALF = 64
NS = 16
CHUNK = B // NS
NQ = 4
QROWS = CHUNK // NQ
RC = 64
NCH = QROWS // RC

ETA_POS = 0.1
ETA_NEG = 1.0
ETA_DECAY = 0.2
RHO_ABSA = 0.75


def _coef_body(inten_ref, conf_ref, pol_ref, addp_ref, lm_ref, ar_ref):
    inten = inten_ref[...]
    conf = conf_ref[...]
    pol = pol_ref[...]
    d = inten * conf
    valid = conf >= RHO_ABSA
    pos = valid & (pol == 1)
    neg = valid & (pol == -1)
    zero = jnp.zeros_like(d)
    addp_ref[...] = jnp.where(pos, ETA_POS * d, zero)
    lm_ref[...] = jnp.where(pos, jnp.log1p(-ETA_DECAY * d), zero)
    ar_ref[...] = jnp.where(neg, ETA_NEG * d, zero)


def _sc_body(pref_hbm, risk_hbm, uid_hbm, kid_hbm, cap_hbm, clm_hbm, car_hbm,
             out_hbm,
             uids, kids, jvec, cap, clm, car,
             uidsc, slotsc, ibuf, vap, vlm, var_,
             pidx, oidx2, widx2, zbuf,
             pbuf, rbuf, obufs, accap, acclm, accar,
             rep_s, aap_s, alm_s, aar_s,
             sem_c, sem_z, sem_a, sem_in, sem_w):
    del sem_c
    c = lax.axis_index("c")
    s = lax.axis_index("s")
    base = s * CHUNK
    chalf = c * HALF
    i16 = lax.iota(jnp.int32, 16)

    d0 = pltpu.async_copy(uid_hbm.at[pl.ds(base, CHUNK)], uids, sem_z)
    d1 = pltpu.async_copy(kid_hbm.at[pl.ds(base, CHUNK)], kids, sem_z)
    d2 = pltpu.async_copy(cap_hbm.at[pl.ds(base, CHUNK)], cap, sem_z)
    d3 = pltpu.async_copy(clm_hbm.at[pl.ds(base, CHUNK)], clm, sem_z)
    d4 = pltpu.async_copy(car_hbm.at[pl.ds(base, CHUNK)], car, sem_z)

    def _mk_jvec(g, carry):
        jvec[pl.ds(g * 16, 16)] = base + g * 16 + i16
        return carry
    lax.fori_loop(0, CHUNK // 16, _mk_jvec, 0)

    def _mk_zbuf(g, carry):
        zbuf[pl.ds(g * 16, 16)] = jnp.zeros((16,), jnp.float32)
        return carry
    lax.fori_loop(0, (RC * HALF) // 16, _mk_zbuf, 0)
    for dd in (d0, d1, d2, d3, d4):
        dd.wait()

    for q in range(8):
        for g in range(8):
            uidsc[q, pl.ds(g * 16, 16)] = uids[pl.ds(q * 128 + g * 16, 16)]

    for q in range(8):
        pltpu.sync_copy(jvec.at[pl.ds(q * 128, 128)], rep_s.at[uidsc.at[q]])
    plsc.subcore_barrier()

    for q in range(8):
        pltpu.sync_copy(rep_s.at[uidsc.at[q]], slotsc.at[q])

    def slots_at(pos, n16):
        return slotsc[pos // 128, pl.ds(pos % 128, 16)]

    wr_pend = []

    def drain_writes(keep):
        while len(wr_pend) > keep:
            wr_pend.pop(0)[0].wait()

    for q in range(NQ):
        zds = []
        for t in range(NCH):
            off = (s * QROWS + t * RC) * HALF
            zds.append(pltpu.async_copy(
                zbuf, aap_s.at[pl.ds(off, RC * HALF)], sem_z))
            zds.append(pltpu.async_copy(
                zbuf, alm_s.at[pl.ds(off, RC * HALF)], sem_z))
            zds.append(pltpu.async_copy(
                zbuf, aar_s.at[pl.ds(off, RC * HALF)], sem_z))
        for dd in zds:
            dd.wait()
        plsc.subcore_barrier()

        def _fill(gg, carry):
            row = gg // 8
            col = (gg % 8) * 16
            k = kids[pl.ds(gg * 16, 16)]
            sl = slots_dyn = slotsc[row, pl.ds(col, 16)]
            apv = cap[pl.ds(gg * 16, 16)]
            lmv = clm[pl.ds(gg * 16, 16)]
            arv = car[pl.ds(gg * 16, 16)]
            kp = k - chalf
            inq = ((sl & 3) == q) & (kp >= 0) & (kp < HALF)
            kpc = jnp.clip(kp, 0, HALF - 1)
            pos = (sl >> 2) * HALF + kpc
            zero = jnp.zeros_like(apv)
            ibuf[row, pl.ds(col, 16)] = pos
            vap[row, pl.ds(col, 16)] = jnp.where(inq, apv, zero)
            vlm[row, pl.ds(col, 16)] = jnp.where(inq, lmv, zero)
            var_[row, pl.ds(col, 16)] = jnp.where(inq, arv, zero)
            return carry
        lax.fori_loop(0, CHUNK // 16, _fill, 0)

        for g in range(8):
            pltpu.sync_copy(vap.at[g], aap_s.at[ibuf.at[g]], add=True)
            pltpu.sync_copy(vlm.at[g], alm_s.at[ibuf.at[g]], add=True)
            pltpu.sync_copy(var_.at[g], aar_s.at[ibuf.at[g]], add=True)
        plsc.subcore_barrier()

        for t in range(NCH):
            tb = t & 1
            drain_writes(1)
            for g in range(RC // 16):
                m = t * RC + g * 16 + i16
                jl = 4 * m + q
                pidx[pl.ds(g * 16, 16)] = plsc.load_gather(uids, [jl])
                oidx2[tb, pl.ds(g * 16, 16)] = (base + jl) * 2 + c
            off = (s * QROWS + t * RC) * HALF
            g0 = pltpu.async_copy(pref_hbm.at[pidx], pbuf, sem_in)
            g1 = pltpu.async_copy(risk_hbm.at[pidx], rbuf, sem_in)
            g2 = pltpu.async_copy(aap_s.at[pl.ds(off, RC * HALF)],
                                  accap, sem_z)
            g3 = pltpu.async_copy(alm_s.at[pl.ds(off, RC * HALF)],
                                  acclm, sem_z)
            g4 = pltpu.async_copy(aar_s.at[pl.ds(off, RC * HALF)],
                                  accar, sem_z)
            for dd in (g0, g1, g2, g3, g4):
                dd.wait()

            def _row(r, carry):
                for g in range(4):
                    aap = accap[pl.ds(r * HALF + g * 16, 16)]
                    alm = acclm[pl.ds(r * HALF + g * 16, 16)]
                    aar = accar[pl.ds(r * HALF + g * 16, 16)]
                    pv = pbuf[r, pl.ds(chalf + g * 16, 16)]
                    rv = rbuf[r, pl.ds(chalf + g * 16, 16)]
                    obufs[tb, r, pl.ds(g * 16, 16)] = jnp.clip(
                        pv + aap, 0.0, 1.0)
                    obufs[tb, r, pl.ds(HALF + g * 16, 16)] = jnp.clip(
                        rv * jnp.exp(alm) + aar, 0.0, 1.0)
                return carry
            lax.fori_loop(0, RC, _row, 0)
            wr_pend.append((pltpu.async_copy(
                obufs.at[tb], out_hbm.at[oidx2.at[tb]], sem_w), tb))
    drain_writes(0)
    plsc.subcore_barrier()

    rd_pend = []
    for t in range(CHUNK // RC):
        tb = t & 1
        buf = pbuf if tb == 0 else rbuf
        if len(rd_pend) >= 2:
            rd_pend.pop(0).wait()
        for g in range(RC // 16):
            pos = t * RC + g * 16
            sl = slots_at(pos, 16)
            widx2[tb, pl.ds(g * 16, 16)] = sl * 2 + c
            oidx2[tb, pl.ds(g * 16, 16)] = (base + pos + i16) * 2 + c
        pltpu.async_copy(out_hbm.at[widx2.at[tb]], buf, sem_in).wait()
        rd_pend.append(pltpu.async_copy(
            buf, out_hbm.at[oidx2.at[tb]], sem_w))
    for dd in rd_pend:
        dd.wait()


def kernel(pref, risk, intensity, confidence, user_ids, attr_ids, polarity):
    addp, lm, ar = pl.pallas_call(
        _coef_body,
        out_shape=(jax.ShapeDtypeStruct((128, 128), jnp.float32),) * 3,
    )(intensity.reshape(128, 128), confidence.reshape(128, 128),
      polarity.reshape(128, 128))

    mesh = plsc.VectorSubcoreMesh(core_axis_name="c", subcore_axis_name="s")
    fn = pl.kernel(
        _sc_body,
        out_type=jax.ShapeDtypeStruct((B * 2, A), jnp.float32),
        mesh=mesh,
        compiler_params=pltpu.CompilerParams(needs_layout_passes=False),
        scratch_types=[
            pltpu.VMEM((CHUNK,), jnp.int32),
            pltpu.VMEM((CHUNK,), jnp.int32),
            pltpu.VMEM((CHUNK,), jnp.int32),
            pltpu.VMEM((CHUNK,), jnp.float32),
            pltpu.VMEM((CHUNK,), jnp.float32),
            pltpu.VMEM((CHUNK,), jnp.float32),
            pltpu.VMEM((8, 128), jnp.int32),
            pltpu.VMEM((8, 128), jnp.int32),
            pltpu.VMEM((8, 128), jnp.int32),
            pltpu.VMEM((8, 128), jnp.float32),
            pltpu.VMEM((8, 128), jnp.float32),
            pltpu.VMEM((8, 128), jnp.float32),
            pltpu.VMEM((RC,), jnp.int32),
            pltpu.VMEM((2, RC), jnp.int32),
            pltpu.VMEM((2, RC), jnp.int32),
            pltpu.VMEM((RC * HALF,), jnp.float32),
            pltpu.VMEM((RC, A), jnp.float32),
            pltpu.VMEM((RC, A), jnp.float32),
            pltpu.VMEM((2, RC, A), jnp.float32),
            pltpu.VMEM((RC * HALF,), jnp.float32),
            pltpu.VMEM((RC * HALF,), jnp.float32),
            pltpu.VMEM((RC * HALF,), jnp.float32),
            pltpu.VMEM_SHARED((U,), jnp.int32),
            pltpu.VMEM_SHARED((B // NQ * HALF,), jnp.float32),
            pltpu.VMEM_SHARED((B // NQ * HALF,), jnp.float32),
            pltpu.VMEM_SHARED((B // NQ * HALF,), jnp.float32),
            pltpu.SemaphoreType.DMA,
            pltpu.SemaphoreType.DMA,
            pltpu.SemaphoreType.DMA,
            pltpu.SemaphoreType.DMA,
            pltpu.SemaphoreType.DMA,
        ],
    )
    out2 = fn(pref, risk, user_ids, attr_ids,
              addp.reshape(B), lm.reshape(B), ar.reshape(B))
    return jnp.swapaxes(out2.reshape(B, 2, 2, HALF), 1, 2).reshape(B, 2 * A)


# speedup vs baseline: 2.0991x; 1.4945x over previous
import jax
import jax.numpy as jnp
from jax import lax
from jax.experimental import pallas as pl
from jax.experimental.pallas import tpu as pltpu
from jax.experimental.pallas import tpu_sc as plsc

U = 100000
A = 128
B = 16384
NS = 16
CHUNK = B // NS
NP = 4
PROWS = CHUNK // 8
RC = 32
NCH = PROWS // RC

ETA_POS = 0.1
ETA_NEG = 1.0
ETA_DECAY = 0.2
RHO_ABSA = 0.75


def _coef_body(inten_ref, conf_ref, pol_ref, addp_ref, lm_ref, ar_ref):
    inten = inten_ref[...]
    conf = conf_ref[...]
    pol = pol_ref[...]
    d = inten * conf
    valid = conf >= RHO_ABSA
    pos = valid & (pol == 1)
    neg = valid & (pol == -1)
    zero = jnp.zeros_like(d)
    addp_ref[...] = jnp.where(pos, ETA_POS * d, zero)
    lm_ref[...] = jnp.where(pos, jnp.log1p(-ETA_DECAY * d), zero)
    ar_ref[...] = jnp.where(neg, ETA_NEG * d, zero)


def _sc_body(pref_hbm, risk_hbm, uid_hbm, kid_hbm, cap_hbm, clm_hbm, car_hbm,
             out_hbm,
             uids, kids, jvec, cap, clm, car,
             uidsc, slotsc, ibuf, vap, vlm, var_,
             pidx, oidx2, widx2, zbuf,
             pbuf, rbuf, obufs, pullbufs, accap, acclm, accar,
             rep_s, aap_s, alm_s, aar_s,
             sem_z, sem_a, sem_in, sem_w):
    c = lax.axis_index("c")
    s = lax.axis_index("s")
    base = s * CHUNK
    i16 = lax.iota(jnp.int32, 16)

    d0 = pltpu.async_copy(uid_hbm.at[pl.ds(base, CHUNK)], uids, sem_z)
    d1 = pltpu.async_copy(kid_hbm.at[pl.ds(base, CHUNK)], kids, sem_z)
    d2 = pltpu.async_copy(cap_hbm.at[pl.ds(base, CHUNK)], cap, sem_z)
    d3 = pltpu.async_copy(clm_hbm.at[pl.ds(base, CHUNK)], clm, sem_z)
    d4 = pltpu.async_copy(car_hbm.at[pl.ds(base, CHUNK)], car, sem_z)

    def _mk_jvec(g, carry):
        jvec[pl.ds(g * 16, 16)] = base + g * 16 + i16
        return carry
    lax.fori_loop(0, CHUNK // 16, _mk_jvec, 0)

    def _mk_zbuf(g, carry):
        zbuf[pl.ds(g * 16, 16)] = jnp.zeros((16,), jnp.float32)
        return carry
    lax.fori_loop(0, (RC * A) // 16, _mk_zbuf, 0)
    for dd in (d0, d1, d2, d3, d4):
        dd.wait()

    for q in range(8):
        for g in range(8):
            uidsc[q, pl.ds(g * 16, 16)] = uids[pl.ds(q * 128 + g * 16, 16)]

    for q in range(8):
        pltpu.sync_copy(jvec.at[pl.ds(q * 128, 128)], rep_s.at[uidsc.at[q]])
    plsc.subcore_barrier()

    for q in range(8):
        pltpu.sync_copy(rep_s.at[uidsc.at[q]], slotsc.at[q])

    wr_pend = []

    def drain_writes(keep):
        while len(wr_pend) > keep:
            wr_pend.pop(0).wait()

    for q in range(NP):
        part = 2 * q + c

        zds = []
        for t in range(NCH):
            off = (s * PROWS + t * RC) * A
            for acc_s in (aap_s, alm_s, aar_s):
                zds.append(pltpu.async_copy(
                    zbuf, acc_s.at[pl.ds(off, RC * A)], sem_z))
        for dd in zds:
            dd.wait()
        plsc.subcore_barrier()

        def _fill(gg, carry):
            row = gg // 8
            col = (gg % 8) * 16
            k = kids[pl.ds(gg * 16, 16)]
            sl = slotsc[row, pl.ds(col, 16)]
            apv = cap[pl.ds(gg * 16, 16)]
            lmv = clm[pl.ds(gg * 16, 16)]
            arv = car[pl.ds(gg * 16, 16)]
            inq = (sl & 7) == part
            pos = (sl >> 3) * A + k
            zero = jnp.zeros_like(apv)
            ibuf[row, pl.ds(col, 16)] = pos
            vap[row, pl.ds(col, 16)] = jnp.where(inq, apv, zero)
            vlm[row, pl.ds(col, 16)] = jnp.where(inq, lmv, zero)
            var_[row, pl.ds(col, 16)] = jnp.where(inq, arv, zero)
            return carry
        lax.fori_loop(0, CHUNK // 16, _fill, 0)

        for g in range(8):
            pltpu.sync_copy(vap.at[g], aap_s.at[ibuf.at[g]], add=True)
            pltpu.sync_copy(vlm.at[g], alm_s.at[ibuf.at[g]], add=True)
            pltpu.sync_copy(var_.at[g], aar_s.at[ibuf.at[g]], add=True)
        plsc.subcore_barrier()

        for t in range(NCH):
            tb = t & 1
            drain_writes(1)
            for g in range(RC // 16):
                m = t * RC + g * 16 + i16
                jl = 8 * m + part
                pidx[pl.ds(g * 16, 16)] = plsc.load_gather(uids, [jl])
                oidx2[tb, pl.ds(g * 16, 16)] = base + jl
            off = (s * PROWS + t * RC) * A
            g0 = pltpu.async_copy(pref_hbm.at[pidx], pbuf, sem_in)
            g1 = pltpu.async_copy(risk_hbm.at[pidx], rbuf, sem_in)
            g2 = pltpu.async_copy(aap_s.at[pl.ds(off, RC * A)], accap, sem_z)
            g3 = pltpu.async_copy(alm_s.at[pl.ds(off, RC * A)], acclm, sem_z)
            g4 = pltpu.async_copy(aar_s.at[pl.ds(off, RC * A)], accar, sem_z)
            for dd in (g0, g1, g2, g3, g4):
                dd.wait()

            def _row(r, carry):
                for g in range(8):
                    aap = accap[pl.ds(r * A + g * 16, 16)]
                    alm = acclm[pl.ds(r * A + g * 16, 16)]
                    aar = accar[pl.ds(r * A + g * 16, 16)]
                    pv = pbuf[r, pl.ds(g * 16, 16)]
                    rv = rbuf[r, pl.ds(g * 16, 16)]
                    obufs[tb, r, pl.ds(g * 16, 16)] = jnp.clip(
                        pv + aap, 0.0, 1.0)
                    obufs[tb, r, pl.ds(A + g * 16, 16)] = jnp.clip(
                        rv * jnp.exp(alm) + aar, 0.0, 1.0)
                return carry
            lax.fori_loop(0, RC, _row, 0)
            wr_pend.append(pltpu.async_copy(
                obufs.at[tb], out_hbm.at[oidx2.at[tb]], sem_w))
    drain_writes(0)
    plsc.subcore_barrier()

    gd_pend = []
    for t in range(CHUNK // RC):
        tb = t & 1
        drain_writes(0)
        for g in range(RC // 16):
            pos = t * RC + g * 16
            sl = slotsc[pos // 128, pl.ds(pos % 128, 16)]
            iv = base + pos + i16
            mine = (sl & 1) == c
            dump = B + s * 8 + (iv & 7)
            widx2[tb, pl.ds(g * 16, 16)] = sl
            oidx2[tb, pl.ds(g * 16, 16)] = jnp.where(mine, iv, dump)
        gd_pend.append((pltpu.async_copy(
            out_hbm.at[widx2.at[tb]], pullbufs.at[tb], sem_in), tb))
        if len(gd_pend) == 2:
            dd, db = gd_pend.pop(0)
            dd.wait()
            wr_pend.append(pltpu.async_copy(
                pullbufs.at[db], out_hbm.at[oidx2.at[db]], sem_w))
    for dd, db in gd_pend:
        dd.wait()
        wr_pend.append(pltpu.async_copy(
            pullbufs.at[db], out_hbm.at[oidx2.at[db]], sem_w))
    drain_writes(0)


def kernel(pref, risk, intensity, confidence, user_ids, attr_ids, polarity):
    addp, lm, ar = pl.pallas_call(
        _coef_body,
        out_shape=(jax.ShapeDtypeStruct((128, 128), jnp.float32),) * 3,
    )(intensity.reshape(128, 128), confidence.reshape(128, 128),
      polarity.reshape(128, 128))

    mesh = plsc.VectorSubcoreMesh(core_axis_name="c", subcore_axis_name="s")
    fn = pl.kernel(
        _sc_body,
        out_type=jax.ShapeDtypeStruct((B + 128, 2 * A), jnp.float32),
        mesh=mesh,
        compiler_params=pltpu.CompilerParams(needs_layout_passes=False),
        scratch_types=[
            pltpu.VMEM((CHUNK,), jnp.int32),
            pltpu.VMEM((CHUNK,), jnp.int32),
            pltpu.VMEM((CHUNK,), jnp.int32),
            pltpu.VMEM((CHUNK,), jnp.float32),
            pltpu.VMEM((CHUNK,), jnp.float32),
            pltpu.VMEM((CHUNK,), jnp.float32),
            pltpu.VMEM((8, 128), jnp.int32),
            pltpu.VMEM((8, 128), jnp.int32),
            pltpu.VMEM((8, 128), jnp.int32),
            pltpu.VMEM((8, 128), jnp.float32),
            pltpu.VMEM((8, 128), jnp.float32),
            pltpu.VMEM((8, 128), jnp.float32),
            pltpu.VMEM((RC,), jnp.int32),
            pltpu.VMEM((2, RC), jnp.int32),
            pltpu.VMEM((2, RC), jnp.int32),
            pltpu.VMEM((RC * A,), jnp.float32),
            pltpu.VMEM((RC, A), jnp.float32),
            pltpu.VMEM((RC, A), jnp.float32),
            pltpu.VMEM((2, RC, 2 * A), jnp.float32),
            pltpu.VMEM((2, RC, 2 * A), jnp.float32),
            pltpu.VMEM((RC * A,), jnp.float32),
            pltpu.VMEM((RC * A,), jnp.float32),
            pltpu.VMEM((RC * A,), jnp.float32),
            pltpu.VMEM_SHARED((U,), jnp.int32),
            pltpu.VMEM_SHARED((B // 8 * A,), jnp.float32),
            pltpu.VMEM_SHARED((B // 8 * A,), jnp.float32),
            pltpu.VMEM_SHARED((B // 8 * A,), jnp.float32),
            pltpu.SemaphoreType.DMA,
            pltpu.SemaphoreType.DMA,
            pltpu.SemaphoreType.DMA,
            pltpu.SemaphoreType.DMA,
        ],
    )
    return fn(pref, risk, user_ids, attr_ids,
              addp.reshape(B), lm.reshape(B), ar.reshape(B))[:B]


# speedup vs baseline: 2.1639x; 1.0309x over previous
import jax
import jax.numpy as jnp
from jax import lax
from jax.experimental import pallas as pl
from jax.experimental.pallas import tpu as pltpu
from jax.experimental.pallas import tpu_sc as plsc

U = 100000
A = 128
B = 16384
NS = 16
CHUNK = B // NS
NP = 4
PROWS = CHUNK // 8
RC = 32
NCH = PROWS // RC

ETA_POS = 0.1
ETA_NEG = 1.0
ETA_DECAY = 0.2
RHO_ABSA = 0.75


def _coef_body(inten_ref, conf_ref, pol_ref, addp_ref, lm_ref, ar_ref):
    inten = inten_ref[...]
    conf = conf_ref[...]
    pol = pol_ref[...]
    d = inten * conf
    valid = conf >= RHO_ABSA
    pos = valid & (pol == 1)
    neg = valid & (pol == -1)
    zero = jnp.zeros_like(d)
    addp_ref[...] = jnp.where(pos, ETA_POS * d, zero)
    lm_ref[...] = jnp.where(pos, jnp.log1p(-ETA_DECAY * d), zero)
    ar_ref[...] = jnp.where(neg, ETA_NEG * d, zero)


def _sc_body(pref_hbm, risk_hbm, uid_hbm, kid_hbm, cap_hbm, clm_hbm, car_hbm,
             out_hbm,
             uids, kids, jvec, cap, clm, car,
             uidsc, slotsc, ibuf, vap, vlm, var_,
             pidx, oidx2, widx2, zbuf,
             pbuf, rbuf, obufs, pullbufs, accap, acclm, accar,
             rep_s, aap_s, alm_s, aar_s,
             sem_z, sem_a, sem_in, sem_w):
    c = lax.axis_index("c")
    s = lax.axis_index("s")
    base = s * CHUNK
    i16 = lax.iota(jnp.int32, 16)

    d0 = pltpu.async_copy(uid_hbm.at[pl.ds(base, CHUNK)], uids, sem_z)
    d1 = pltpu.async_copy(kid_hbm.at[pl.ds(base, CHUNK)], kids, sem_z)
    d2 = pltpu.async_copy(cap_hbm.at[pl.ds(base, CHUNK)], cap, sem_z)
    d3 = pltpu.async_copy(clm_hbm.at[pl.ds(base, CHUNK)], clm, sem_z)
    d4 = pltpu.async_copy(car_hbm.at[pl.ds(base, CHUNK)], car, sem_z)

    def _mk_jvec(g, carry):
        jvec[pl.ds(g * 16, 16)] = base + g * 16 + i16
        return carry
    lax.fori_loop(0, CHUNK // 16, _mk_jvec, 0)

    def _mk_zbuf(g, carry):
        zbuf[pl.ds(g * 16, 16)] = jnp.zeros((16,), jnp.float32)
        return carry
    lax.fori_loop(0, (RC * A) // 16, _mk_zbuf, 0)
    for dd in (d0, d1, d2, d3, d4):
        dd.wait()

    for q in range(8):
        for g in range(8):
            uidsc[q, pl.ds(g * 16, 16)] = uids[pl.ds(q * 128 + g * 16, 16)]

    for q in range(8):
        pltpu.sync_copy(jvec.at[pl.ds(q * 128, 128)], rep_s.at[uidsc.at[q]])
    plsc.subcore_barrier()

    for q in range(8):
        pltpu.sync_copy(rep_s.at[uidsc.at[q]], slotsc.at[q])

    wr_pend = []

    def drain_writes(keep):
        while len(wr_pend) > keep:
            wr_pend.pop(0).wait()

    for q in range(NP):
        part = 2 * q + c

        zds = []
        for t in range(NCH):
            off = (s * PROWS + t * RC) * A
            for acc_s in (aap_s, alm_s, aar_s):
                zds.append(pltpu.async_copy(
                    zbuf, acc_s.at[pl.ds(off, RC * A)], sem_z))

        def _fill(gg, carry):
            row = gg // 8
            col = (gg % 8) * 16
            k = kids[pl.ds(gg * 16, 16)]
            sl = slotsc[row, pl.ds(col, 16)]
            apv = cap[pl.ds(gg * 16, 16)]
            lmv = clm[pl.ds(gg * 16, 16)]
            arv = car[pl.ds(gg * 16, 16)]
            inq = (sl & 7) == part
            pos = (sl >> 3) * A + k
            zero = jnp.zeros_like(apv)
            ibuf[row, pl.ds(col, 16)] = pos
            vap[row, pl.ds(col, 16)] = jnp.where(inq, apv, zero)
            vlm[row, pl.ds(col, 16)] = jnp.where(inq, lmv, zero)
            var_[row, pl.ds(col, 16)] = jnp.where(inq, arv, zero)
            return carry
        lax.fori_loop(0, CHUNK // 16, _fill, 0)
        for dd in zds:
            dd.wait()
        plsc.subcore_barrier()

        ads = []
        for g in range(8):
            ads.append(pltpu.async_copy(
                vap.at[g], aap_s.at[ibuf.at[g]], sem_a, add=True))
            ads.append(pltpu.async_copy(
                vlm.at[g], alm_s.at[ibuf.at[g]], sem_a, add=True))
            ads.append(pltpu.async_copy(
                var_.at[g], aar_s.at[ibuf.at[g]], sem_a, add=True))
        for dd in ads:
            dd.wait()
        plsc.subcore_barrier()

        for t in range(NCH):
            tb = t & 1
            drain_writes(1)
            for g in range(RC // 16):
                m = t * RC + g * 16 + i16
                jl = 8 * m + part
                pidx[pl.ds(g * 16, 16)] = plsc.load_gather(uids, [jl])
                oidx2[tb, pl.ds(g * 16, 16)] = base + jl
            off = (s * PROWS + t * RC) * A
            g0 = pltpu.async_copy(pref_hbm.at[pidx], pbuf, sem_in)
            g1 = pltpu.async_copy(risk_hbm.at[pidx], rbuf, sem_in)
            g2 = pltpu.async_copy(aap_s.at[pl.ds(off, RC * A)], accap, sem_z)
            g3 = pltpu.async_copy(alm_s.at[pl.ds(off, RC * A)], acclm, sem_z)
            g4 = pltpu.async_copy(aar_s.at[pl.ds(off, RC * A)], accar, sem_z)
            for dd in (g0, g1, g2, g3, g4):
                dd.wait()

            def _row(r, carry):
                for g in range(8):
                    aap = accap[pl.ds(r * A + g * 16, 16)]
                    alm = acclm[pl.ds(r * A + g * 16, 16)]
                    aar = accar[pl.ds(r * A + g * 16, 16)]
                    pv = pbuf[r, pl.ds(g * 16, 16)]
                    rv = rbuf[r, pl.ds(g * 16, 16)]
                    obufs[tb, r, pl.ds(g * 16, 16)] = jnp.clip(
                        pv + aap, 0.0, 1.0)
                    obufs[tb, r, pl.ds(A + g * 16, 16)] = jnp.clip(
                        rv * jnp.exp(alm) + aar, 0.0, 1.0)
                return carry
            lax.fori_loop(0, RC, _row, 0)
            wr_pend.append(pltpu.async_copy(
                obufs.at[tb], out_hbm.at[oidx2.at[tb]], sem_w))
    drain_writes(0)
    plsc.subcore_barrier()

    gd_pend = []
    for t in range(CHUNK // RC):
        tb = t & 1
        drain_writes(0)
        for g in range(RC // 16):
            pos = t * RC + g * 16
            sl = slotsc[pos // 128, pl.ds(pos % 128, 16)]
            iv = base + pos + i16
            mine = (sl & 1) == c
            dump = B + s * 8 + (iv & 7)
            widx2[tb, pl.ds(g * 16, 16)] = sl
            oidx2[tb, pl.ds(g * 16, 16)] = jnp.where(mine, iv, dump)
        gd_pend.append((pltpu.async_copy(
            out_hbm.at[widx2.at[tb]], pullbufs.at[tb], sem_in), tb))
        if len(gd_pend) == 2:
            dd, db = gd_pend.pop(0)
            dd.wait()
            wr_pend.append(pltpu.async_copy(
                pullbufs.at[db], out_hbm.at[oidx2.at[db]], sem_w))
    for dd, db in gd_pend:
        dd.wait()
        wr_pend.append(pltpu.async_copy(
            pullbufs.at[db], out_hbm.at[oidx2.at[db]], sem_w))
    drain_writes(0)


def kernel(pref, risk, intensity, confidence, user_ids, attr_ids, polarity):
    addp, lm, ar = pl.pallas_call(
        _coef_body,
        out_shape=(jax.ShapeDtypeStruct((128, 128), jnp.float32),) * 3,
    )(intensity.reshape(128, 128), confidence.reshape(128, 128),
      polarity.reshape(128, 128))

    mesh = plsc.VectorSubcoreMesh(core_axis_name="c", subcore_axis_name="s")
    fn = pl.kernel(
        _sc_body,
        out_type=jax.ShapeDtypeStruct((B + 128, 2 * A), jnp.float32),
        mesh=mesh,
        compiler_params=pltpu.CompilerParams(needs_layout_passes=False),
        scratch_types=[
            pltpu.VMEM((CHUNK,), jnp.int32),
            pltpu.VMEM((CHUNK,), jnp.int32),
            pltpu.VMEM((CHUNK,), jnp.int32),
            pltpu.VMEM((CHUNK,), jnp.float32),
            pltpu.VMEM((CHUNK,), jnp.float32),
            pltpu.VMEM((CHUNK,), jnp.float32),
            pltpu.VMEM((8, 128), jnp.int32),
            pltpu.VMEM((8, 128), jnp.int32),
            pltpu.VMEM((8, 128), jnp.int32),
            pltpu.VMEM((8, 128), jnp.float32),
            pltpu.VMEM((8, 128), jnp.float32),
            pltpu.VMEM((8, 128), jnp.float32),
            pltpu.VMEM((RC,), jnp.int32),
            pltpu.VMEM((2, RC), jnp.int32),
            pltpu.VMEM((2, RC), jnp.int32),
            pltpu.VMEM((RC * A,), jnp.float32),
            pltpu.VMEM((RC, A), jnp.float32),
            pltpu.VMEM((RC, A), jnp.float32),
            pltpu.VMEM((2, RC, 2 * A), jnp.float32),
            pltpu.VMEM((2, RC, 2 * A), jnp.float32),
            pltpu.VMEM((RC * A,), jnp.float32),
            pltpu.VMEM((RC * A,), jnp.float32),
            pltpu.VMEM((RC * A,), jnp.float32),
            pltpu.VMEM_SHARED((U,), jnp.int32),
            pltpu.VMEM_SHARED((B // 8 * A,), jnp.float32),
            pltpu.VMEM_SHARED((B // 8 * A,), jnp.float32),
            pltpu.VMEM_SHARED((B // 8 * A,), jnp.float32),
            pltpu.SemaphoreType.DMA,
            pltpu.SemaphoreType.DMA,
            pltpu.SemaphoreType.DMA,
            pltpu.SemaphoreType.DMA,
        ],
    )
    return fn(pref, risk, user_ids, attr_ids,
              addp.reshape(B), lm.reshape(B), ar.reshape(B))[:B]
